# baseline (speedup 1.0000x reference)
import functools
import jax
import jax.numpy as jnp
from jax import lax
from jax.experimental import pallas as pl
from jax.experimental.pallas import tpu as pltpu
from jax.experimental.pallas import tpu_sc as plsc

N = 10000
E = 320000
D = 128
DE = 16

EBLK = 6400
NEB = E // EBLK

NEG = -1e30


def _matmul2_kernel(x_ref, wa_ref, wb_ref, oa_ref, ob_ref):
    x = x_ref[...]
    oa_ref[...] = jnp.dot(x, wa_ref[...], preferred_element_type=jnp.float32, precision='highest')
    ob_ref[...] = jnp.dot(x, wb_ref[...], preferred_element_type=jnp.float32, precision='highest')


def _proj2(x, wa, wb):
    return pl.pallas_call(
        _matmul2_kernel,
        out_shape=(jax.ShapeDtypeStruct((N, D), jnp.float32),
                   jax.ShapeDtypeStruct((N, D), jnp.float32)),
    )(x, wa, wb)


NBANK = 8


def _segmax_kernel(src_ref, tar_ref, ef_ref, pb_ref, c_ref, m_ref, efc_ref,
                   *accs):
    pid = pl.program_id(0)

    @pl.when(pid == 0)
    def _init():
        for a in accs:
            a[...] = jnp.full((N, D), NEG, dtype=jnp.float32)

    efc_ref[...] = jnp.dot(ef_ref[...], c_ref[...],
                           preferred_element_type=jnp.float32)

    def body(i, _):
        e = i * NBANK
        for j, a in enumerate(accs):
            s = src_ref[0, 0, e + j]
            t = tar_ref[0, 0, e + j]
            a[t, :] = jnp.maximum(a[t, :], pb_ref[s, :] + efc_ref[e + j, :])
        return 0

    jax.lax.fori_loop(0, EBLK // NBANK, body, 0, unroll=32)

    @pl.when(pid == NEB - 1)
    def _fin():
        m = accs[0][...]
        for a in accs[1:]:
            m = jnp.maximum(m, a[...])
        m_ref[...] = m


def _segmax(src, tar, ef, pb, c):
    return pl.pallas_call(
        _segmax_kernel,
        grid=(NEB,),
        in_specs=[
            pl.BlockSpec((1, 1, EBLK), lambda i: (i, 0, 0), memory_space=pltpu.SMEM),
            pl.BlockSpec((1, 1, EBLK), lambda i: (i, 0, 0), memory_space=pltpu.SMEM),
            pl.BlockSpec((EBLK, DE), lambda i: (i, 0)),
            pl.BlockSpec((N, D), lambda i: (0, 0)),
            pl.BlockSpec((DE, D), lambda i: (0, 0)),
        ],
        out_specs=pl.BlockSpec((N, D), lambda i: (0, 0)),
        out_shape=jax.ShapeDtypeStruct((N, D), jnp.float32),
        scratch_shapes=[pltpu.VMEM((EBLK, D), jnp.float32)]
        + [pltpu.VMEM((N, D), jnp.float32) for _ in range(NBANK)],
    )(src.reshape(NEB, 1, EBLK), tar.reshape(NEB, 1, EBLK), ef, pb, c)


def _update_kernel(nf_ref, pa_ref, m_ref, b_ref, nfo_ref):
    agg = jnp.maximum(pa_ref[...] + m_ref[...] + b_ref[...], 0.0)
    nfo_ref[...] = nf_ref[...] + agg


def _update(nf, pa, m, b):
    return pl.pallas_call(
        _update_kernel,
        out_shape=jax.ShapeDtypeStruct((N, D), jnp.float32),
    )(nf, pa, m, b.reshape(1, D))


def _nodeout_kernel(nf_ref, w1_ref, b1_ref, w2_ref, b2_ref, o_ref):
    h = jnp.maximum(jnp.dot(nf_ref[...], w1_ref[...],
                            preferred_element_type=jnp.float32, precision='highest') + b1_ref[...], 0.0)
    o_ref[...] = jnp.dot(h, w2_ref[...],
                         preferred_element_type=jnp.float32, precision='highest') + b2_ref[...]


def _nodeout(nf, w1, b1, w2, b2):
    return pl.pallas_call(
        _nodeout_kernel,
        out_shape=jax.ShapeDtypeStruct((N, D), jnp.float32),
    )(nf, w1, b1.reshape(1, D), w2, b2.reshape(1, D))


NW = 32
SCCH = 80
SCB = E // NW // SCCH


def _scgather_build():
    mesh = plsc.VectorSubcoreMesh(core_axis_name="c", subcore_axis_name="s")

    @functools.partial(
        pl.kernel, mesh=mesh,
        out_type=jax.ShapeDtypeStruct((E, D), jnp.float32),
        scratch_types=[
            pltpu.VMEM((SCB, SCCH), jnp.int32),
            pltpu.VMEM((SCB, SCCH), jnp.int32),
            pltpu.VMEM((SCCH, D), jnp.float32),
            pltpu.VMEM((SCCH, D), jnp.float32),
            pltpu.SemaphoreType.DMA,
            pltpu.SemaphoreType.DMA,
        ],
    )
    def scgather(src_hbm, tar_hbm, q_hbm, out_hbm, sbuf, tbuf, qs, qt,
                 sem1, sem2):
        w = lax.axis_index("s") * 2 + lax.axis_index("c")
        pltpu.sync_copy(src_hbm.at[w], sbuf)
        pltpu.sync_copy(tar_hbm.at[w], tbuf)

        def batch(j, carry):
            cp1 = pltpu.async_copy(q_hbm.at[sbuf.at[j]], qs, sem1)
            cp2 = pltpu.async_copy(q_hbm.at[tbuf.at[j]], qt, sem2)
            cp1.wait()
            cp2.wait()

            def row(r, c):
                for k in range(D // 16):
                    qs[r, pl.ds(k * 16, 16)] = (qs[r, pl.ds(k * 16, 16)]
                                                - qt[r, pl.ds(k * 16, 16)])
                return c

            lax.fori_loop(0, SCCH, row, 0)
            off = pl.multiple_of((w * SCB + j) * SCCH, SCCH)
            pltpu.sync_copy(qs, out_hbm.at[pl.ds(off, SCCH)])
            return carry

        lax.fori_loop(0, SCB, batch, 0)

    return scgather


def _edgeout_kernel(ef_ref, g_ref, ce_ref, be_ref, w2_ref, b2_ref, o_ref):
    efc = jnp.dot(ef_ref[...], ce_ref[...], preferred_element_type=jnp.float32)
    he = jnp.maximum(g_ref[...] + efc + be_ref[...], 0.0)
    o_ref[...] = jnp.dot(he, w2_ref[...],
                         preferred_element_type=jnp.float32) + b2_ref[...]


def _edgeout(src, tar, ef, q, ce, be, w2, b2):
    g = _scgather_build()(src.reshape(NW, SCB, SCCH),
                          tar.reshape(NW, SCB, SCCH), q)
    return pl.pallas_call(
        _edgeout_kernel,
        grid=(NEB,),
        in_specs=[
            pl.BlockSpec((EBLK, DE), lambda i: (i, 0)),
            pl.BlockSpec((EBLK, D), lambda i: (i, 0)),
            pl.BlockSpec((DE, D), lambda i: (0, 0)),
            pl.BlockSpec((1, D), lambda i: (0, 0)),
            pl.BlockSpec((D, DE), lambda i: (0, 0)),
            pl.BlockSpec((1, DE), lambda i: (0, 0)),
        ],
        out_specs=pl.BlockSpec((EBLK, DE), lambda i: (i, 0)),
        out_shape=jax.ShapeDtypeStruct((E, DE), jnp.float32),
    )(ef, g, ce, be.reshape(1, D), w2, b2.reshape(1, DE))


@jax.jit
def kernel(node_features, edge_indices, edge_features, W_em, b_em,
           c0_W, c0_b, c1_W, c1_b, no_W1, no_b1, no_W2, no_b2,
           eo_W1, eo_b1, eo_W2, eo_b2):
    src = edge_indices[0]
    tar = edge_indices[1]

    A0, B0, C0w = c0_W[:D], c0_W[D:2 * D], c0_W[2 * D:]
    A1, B1, C1w = c1_W[:D], c1_W[D:2 * D], c1_W[2 * D:]
    _dot = functools.partial(jnp.dot, precision='highest')
    C0 = _dot(W_em, C0w)
    C1 = _dot(W_em, C1w)
    b0p = c0_b + _dot(b_em, C0w)
    b1p = c1_b + _dot(b_em, C1w)
    eoA, eoB = eo_W1[:D], eo_W1[D:]
    Ce = _dot(W_em, eoB)
    bep = eo_b1 + _dot(b_em, eoB)

    nf0 = node_features
    pa0, pb0 = _proj2(nf0, A0 - B0, B0)
    m0 = _segmax(src, tar, edge_features, pb0, C0)
    nf1 = _update(nf0, pa0, m0, b0p)

    pa1, pb1 = _proj2(nf1, A1 - B1, B1)
    m1 = _segmax(src, tar, edge_features, pb1, C1)
    nf2 = _update(nf1, pa1, m1, b1p)

    node_output = _nodeout(nf2, no_W1, no_b1, no_W2, no_b2)

    (q,) = pl.pallas_call(
        lambda x_ref, w_ref, o_ref: o_ref.__setitem__(
            (...,), jnp.dot(x_ref[...], w_ref[...],
                            preferred_element_type=jnp.float32, precision='highest')),
        out_shape=[jax.ShapeDtypeStruct((N, D), jnp.float32)],
    )(nf2, eoA)
    edge_out = _edgeout(src, tar, edge_features, q, Ce, bep, eo_W2, eo_b2)

    return node_output, edge_out


# speedup vs baseline: 1.0695x; 1.0695x over previous
import functools
import jax
import jax.numpy as jnp
from jax import lax
from jax.experimental import pallas as pl
from jax.experimental.pallas import tpu as pltpu
from jax.experimental.pallas import tpu_sc as plsc

N = 10000
E = 320000
D = 128
DE = 16

EBLK = 6400
NEB = E // EBLK

NEG = -1e30


def _matmul2_kernel(x_ref, wa_ref, wb_ref, oa_ref, ob_ref):
    x = x_ref[...]
    oa_ref[...] = jnp.dot(x, wa_ref[...], preferred_element_type=jnp.float32, precision='highest')
    ob_ref[...] = jnp.dot(x, wb_ref[...], preferred_element_type=jnp.float32, precision='highest')


def _proj2(x, wa, wb):
    return pl.pallas_call(
        _matmul2_kernel,
        out_shape=(jax.ShapeDtypeStruct((N, D), jnp.float32),
                   jax.ShapeDtypeStruct((N, D), jnp.float32)),
    )(x, wa, wb)


NBANK = 8


def _segmax_kernel(src_ref, tar_ref, ef_ref, pb_ref, c_ref, m_ref, efc_ref,
                   *accs):
    pid = pl.program_id(0)

    @pl.when(pid == 0)
    def _init():
        for a in accs:
            a[...] = jnp.full((N, D), NEG, dtype=jnp.float32)

    efc_ref[...] = jnp.dot(ef_ref[...], c_ref[...],
                           preferred_element_type=jnp.float32)

    def body(i, _):
        e = i * NBANK
        for j, a in enumerate(accs):
            s = src_ref[0, 0, e + j]
            t = tar_ref[0, 0, e + j]
            a[t, :] = jnp.maximum(a[t, :], pb_ref[s, :] + efc_ref[e + j, :])
        return 0

    jax.lax.fori_loop(0, EBLK // NBANK, body, 0, unroll=16)

    @pl.when(pid == NEB - 1)
    def _fin():
        m = accs[0][...]
        for a in accs[1:]:
            m = jnp.maximum(m, a[...])
        m_ref[...] = m


def _segmax(src, tar, ef, pb, c):
    return pl.pallas_call(
        _segmax_kernel,
        grid=(NEB,),
        in_specs=[
            pl.BlockSpec((1, 1, EBLK), lambda i: (i, 0, 0), memory_space=pltpu.SMEM),
            pl.BlockSpec((1, 1, EBLK), lambda i: (i, 0, 0), memory_space=pltpu.SMEM),
            pl.BlockSpec((EBLK, DE), lambda i: (i, 0)),
            pl.BlockSpec((N, D), lambda i: (0, 0)),
            pl.BlockSpec((DE, D), lambda i: (0, 0)),
        ],
        out_specs=pl.BlockSpec((N, D), lambda i: (0, 0)),
        out_shape=jax.ShapeDtypeStruct((N, D), jnp.float32),
        scratch_shapes=[pltpu.VMEM((EBLK, D), jnp.float32)]
        + [pltpu.VMEM((N, D), jnp.float32) for _ in range(NBANK)],
    )(src.reshape(NEB, 1, EBLK), tar.reshape(NEB, 1, EBLK), ef, pb, c)


def _update_kernel(nf_ref, pa_ref, m_ref, b_ref, nfo_ref):
    agg = jnp.maximum(pa_ref[...] + m_ref[...] + b_ref[...], 0.0)
    nfo_ref[...] = nf_ref[...] + agg


def _update(nf, pa, m, b):
    return pl.pallas_call(
        _update_kernel,
        out_shape=jax.ShapeDtypeStruct((N, D), jnp.float32),
    )(nf, pa, m, b.reshape(1, D))


def _nodeout_kernel(nf_ref, w1_ref, b1_ref, w2_ref, b2_ref, o_ref):
    h = jnp.maximum(jnp.dot(nf_ref[...], w1_ref[...],
                            preferred_element_type=jnp.float32, precision='highest') + b1_ref[...], 0.0)
    o_ref[...] = jnp.dot(h, w2_ref[...],
                         preferred_element_type=jnp.float32, precision='highest') + b2_ref[...]


def _nodeout(nf, w1, b1, w2, b2):
    return pl.pallas_call(
        _nodeout_kernel,
        out_shape=jax.ShapeDtypeStruct((N, D), jnp.float32),
    )(nf, w1, b1.reshape(1, D), w2, b2.reshape(1, D))


NW = 32
SCCH = 40
SCB = E // NW // SCCH


def _scgather_build():
    mesh = plsc.VectorSubcoreMesh(core_axis_name="c", subcore_axis_name="s")

    @functools.partial(
        pl.kernel, mesh=mesh,
        out_type=jax.ShapeDtypeStruct((E, D), jnp.float32),
        scratch_types=[
            pltpu.VMEM((SCB, SCCH), jnp.int32),
            pltpu.VMEM((SCB, SCCH), jnp.int32),
            pltpu.VMEM((2, SCCH, D), jnp.float32),
            pltpu.VMEM((2, SCCH, D), jnp.float32),
            pltpu.VMEM((2, SCCH, D), jnp.float32),
            pltpu.SemaphoreType.DMA,
            pltpu.SemaphoreType.DMA,
            pltpu.SemaphoreType.DMA,
            pltpu.SemaphoreType.DMA,
        ],
    )
    def scgather(src_hbm, tar_hbm, q_hbm, out_hbm, sbuf, tbuf, qs, qt, g,
                 sg0, sg1, so0, so1):
        w = lax.axis_index("s") * 2 + lax.axis_index("c")
        pltpu.sync_copy(src_hbm.at[w], sbuf)
        pltpu.sync_copy(tar_hbm.at[w], tbuf)
        sgs = (sg0, sg1)
        sos = (so0, so1)

        def start_gather(j, slot):
            pltpu.async_copy(q_hbm.at[sbuf.at[j]], qs.at[slot], sgs[slot])
            pltpu.async_copy(q_hbm.at[tbuf.at[j]], qt.at[slot], sgs[slot])

        def turn(j, slot):
            jn = jnp.minimum(j + 1, SCB - 1)
            start_gather(jn, 1 - slot)
            pltpu.make_async_copy(q_hbm.at[sbuf.at[j]], qs.at[slot],
                                  sgs[slot]).wait()
            pltpu.make_async_copy(q_hbm.at[tbuf.at[j]], qt.at[slot],
                                  sgs[slot]).wait()

            def row(r, c):
                for k in range(D // 16):
                    g[slot, r, pl.ds(k * 16, 16)] = (
                        qs[slot, r, pl.ds(k * 16, 16)]
                        - qt[slot, r, pl.ds(k * 16, 16)])
                return c

            lax.fori_loop(0, SCCH, row, 0)
            off = pl.multiple_of((w * SCB + j) * SCCH, SCCH)
            @pl.when(j >= 2)
            def _():
                prev = pl.multiple_of((w * SCB + j - 2) * SCCH, SCCH)
                pltpu.make_async_copy(g.at[slot],
                                      out_hbm.at[pl.ds(prev, SCCH)],
                                      sos[slot]).wait()

            pltpu.async_copy(g.at[slot], out_hbm.at[pl.ds(off, SCCH)],
                             sos[slot])

        start_gather(0, 0)

        def pair(i, carry):
            turn(2 * i, 0)
            turn(2 * i + 1, 1)
            return carry

        lax.fori_loop(0, SCB // 2, pair, 0)
        pltpu.make_async_copy(q_hbm.at[sbuf.at[SCB - 1]], qs.at[0],
                              sg0).wait()
        pltpu.make_async_copy(q_hbm.at[tbuf.at[SCB - 1]], qt.at[0],
                              sg0).wait()
        for slot, j in ((0, SCB - 2), (1, SCB - 1)):
            off = pl.multiple_of((w * SCB + j) * SCCH, SCCH)
            pltpu.make_async_copy(g.at[slot], out_hbm.at[pl.ds(off, SCCH)],
                                  sos[slot]).wait()

    return scgather


def _edgeout_kernel(ef_ref, g_ref, ce_ref, be_ref, w2_ref, b2_ref, o_ref):
    efc = jnp.dot(ef_ref[...], ce_ref[...], preferred_element_type=jnp.float32)
    he = jnp.maximum(g_ref[...] + efc + be_ref[...], 0.0)
    o_ref[...] = jnp.dot(he, w2_ref[...],
                         preferred_element_type=jnp.float32) + b2_ref[...]


def _edgeout(src, tar, ef, q, ce, be, w2, b2):
    g = _scgather_build()(src.reshape(NW, SCB, SCCH),
                          tar.reshape(NW, SCB, SCCH), q)
    return pl.pallas_call(
        _edgeout_kernel,
        grid=(NEB,),
        in_specs=[
            pl.BlockSpec((EBLK, DE), lambda i: (i, 0)),
            pl.BlockSpec((EBLK, D), lambda i: (i, 0)),
            pl.BlockSpec((DE, D), lambda i: (0, 0)),
            pl.BlockSpec((1, D), lambda i: (0, 0)),
            pl.BlockSpec((D, DE), lambda i: (0, 0)),
            pl.BlockSpec((1, DE), lambda i: (0, 0)),
        ],
        out_specs=pl.BlockSpec((EBLK, DE), lambda i: (i, 0)),
        out_shape=jax.ShapeDtypeStruct((E, DE), jnp.float32),
    )(ef, g, ce, be.reshape(1, D), w2, b2.reshape(1, DE))


@jax.jit
def kernel(node_features, edge_indices, edge_features, W_em, b_em,
           c0_W, c0_b, c1_W, c1_b, no_W1, no_b1, no_W2, no_b2,
           eo_W1, eo_b1, eo_W2, eo_b2):
    src = edge_indices[0]
    tar = edge_indices[1]

    A0, B0, C0w = c0_W[:D], c0_W[D:2 * D], c0_W[2 * D:]
    A1, B1, C1w = c1_W[:D], c1_W[D:2 * D], c1_W[2 * D:]
    _dot = functools.partial(jnp.dot, precision='highest')
    C0 = _dot(W_em, C0w)
    C1 = _dot(W_em, C1w)
    b0p = c0_b + _dot(b_em, C0w)
    b1p = c1_b + _dot(b_em, C1w)
    eoA, eoB = eo_W1[:D], eo_W1[D:]
    Ce = _dot(W_em, eoB)
    bep = eo_b1 + _dot(b_em, eoB)

    nf0 = node_features
    pa0, pb0 = _proj2(nf0, A0 - B0, B0)
    m0 = _segmax(src, tar, edge_features, pb0, C0)
    nf1 = _update(nf0, pa0, m0, b0p)

    pa1, pb1 = _proj2(nf1, A1 - B1, B1)
    m1 = _segmax(src, tar, edge_features, pb1, C1)
    nf2 = _update(nf1, pa1, m1, b1p)

    node_output = _nodeout(nf2, no_W1, no_b1, no_W2, no_b2)

    (q,) = pl.pallas_call(
        lambda x_ref, w_ref, o_ref: o_ref.__setitem__(
            (...,), jnp.dot(x_ref[...], w_ref[...],
                            preferred_element_type=jnp.float32, precision='highest')),
        out_shape=[jax.ShapeDtypeStruct((N, D), jnp.float32)],
    )(nf2, eoA)
    edge_out = _edgeout(src, tar, edge_features, q, Ce, bep, eo_W2, eo_b2)

    return node_output, edge_out
